# baseline (speedup 1.0000x reference)
import jax
import jax.numpy as jnp
from jax import lax
from jax.experimental import pallas as pl
from jax.experimental.pallas import tpu as pltpu
from jax.experimental.pallas import tpu_sc as plsc

_NUM_TOKENS = 20
_EMBEDDING_DIM = 64
_ROW = _NUM_TOKENS * _EMBEDDING_DIM


def _gather_row_body(idx_hbm, mem_hbm, out_hbm, idx_v, row_v, sem):
    first = (lax.axis_index("c") == 0) & (lax.axis_index("s") == 0)

    @pl.when(first)
    def _():
        pltpu.sync_copy(idx_hbm, idx_v)
        pltpu.async_copy(mem_hbm.at[idx_v], row_v, sem).wait()
        pltpu.sync_copy(row_v, out_hbm)


def kernel(memory, forgetting_factor, task_id):
    del forgetting_factor
    idx = jnp.asarray(task_id, jnp.int32).reshape(1)
    mesh = plsc.VectorSubcoreMesh(core_axis_name="c", subcore_axis_name="s")
    run = pl.kernel(
        _gather_row_body,
        out_type=jax.ShapeDtypeStruct((1, _ROW), jnp.float32),
        mesh=mesh,
        scratch_types=[
            pltpu.VMEM((1,), jnp.int32),
            pltpu.VMEM((1, _ROW), jnp.float32),
            pltpu.SemaphoreType.DMA,
        ],
    )
    out = run(idx, memory)
    return out.reshape(_NUM_TOKENS, _EMBEDDING_DIM)


# speedup vs baseline: 1.1800x; 1.1800x over previous
import functools

import jax
import jax.numpy as jnp
from jax.experimental import pallas as pl
from jax.experimental.pallas import tpu as pltpu
from jax.experimental.pallas import tpu_sc as plsc

_NUM_TOKENS = 20
_EMBEDDING_DIM = 64
_ROW = _NUM_TOKENS * _EMBEDDING_DIM


def kernel(memory, forgetting_factor, task_id):
    del forgetting_factor
    tid = jnp.asarray(task_id, jnp.int32)
    mesh = plsc.ScalarSubcoreMesh(axis_name="a", num_cores=1)

    @functools.partial(
        pl.kernel,
        out_type=jax.ShapeDtypeStruct((1, _ROW), jnp.float32),
        mesh=mesh,
        scratch_types=[pltpu.SemaphoreType.DMA],
    )
    def run(mem_hbm, out_hbm, sem):
        pltpu.async_copy(mem_hbm.at[pl.ds(tid, 1)], out_hbm, sem).wait()

    out = run(memory)
    return out.reshape(_NUM_TOKENS, _EMBEDDING_DIM)
